# baseline (speedup 1.0000x reference)
import functools

import jax
import jax.numpy as jnp
from jax import lax
from jax.experimental import pallas as pl
from jax.experimental.pallas import tpu as pltpu
from jax.experimental.pallas import tpu_sc as plsc


def _topk_idx_body(scores_ref, idxp_ref, idxe_ref):
    N = scores_ref.shape[-1]
    K = idxp_ref.shape[-1]
    b = pl.program_id(0)
    s = scores_ref[0]
    scol = jnp.reshape(s, (N, 1))
    ni = lax.broadcasted_iota(jnp.int32, (N, N), 0)
    mi = lax.broadcasted_iota(jnp.int32, (N, N), 1)
    beats = (s > scol) | ((s == scol) & (mi < ni))
    ones = jnp.ones((N, 8), jnp.float32)
    rank_f = lax.dot_general(beats.astype(jnp.float32), ones,
                             dimension_numbers=(((1,), (0,)), ((), ())))
    rank = rank_f[:, :1].astype(jnp.int32)
    jrow = lax.broadcasted_iota(jnp.int32, (1, K), 1)
    sel = rank == jrow
    nidx = lax.broadcasted_iota(jnp.int32, (N, K), 0)
    idx = jnp.sum(jnp.where(sel, nidx, 0), axis=0, keepdims=True)
    idxp_ref[0] = idx + b * N
    idxe_ref[0] = idx + 1


def _topk_indices(scores, B, N, K):
    scores3 = scores.reshape(B, 1, N)
    idxp, idxe = pl.pallas_call(
        _topk_idx_body,
        grid=(B,),
        in_specs=[pl.BlockSpec((1, 1, N), lambda b: (b, 0, 0))],
        out_specs=[
            pl.BlockSpec((1, 1, K), lambda b: (b, 0, 0)),
            pl.BlockSpec((1, 1, K), lambda b: (b, 0, 0)),
        ],
        out_shape=[
            jax.ShapeDtypeStruct((B, 1, K), jnp.int32),
            jax.ShapeDtypeStruct((B, 1, K), jnp.int32),
        ],
    )(scores3)
    return idxp.reshape(B * K // 32, 32), idxe.reshape(B * K // 32, 32)


_NC, _NS = 2, 16
_NW = _NC * _NS
_CHUNK = 32
_NBUF = 2


def _sc_gather_body(magno_hbm, pos_hbm, idxp_hbm, idxe_hbm, out_hbm,
                    idxp_v, idxe_v, rows_v, pose_v, gsems, ssems):
    D = rows_v[0].shape[-1]
    K = out_hbm.shape[1]
    rows_total = out_hbm.shape[0] * K
    rows_per_w = rows_total // _NW

    def out_slice(c):
        off = base + c * _CHUNK
        return out_hbm.at[off // K, pl.ds(pl.multiple_of(off % K, _CHUNK),
                                          _CHUNK)]
    nchunk = rows_per_w // _CHUNK
    wid = lax.axis_index("s") * _NC + lax.axis_index("c")
    base = wid * rows_per_w
    cbase = wid * nchunk

    pltpu.sync_copy(idxp_hbm.at[pl.ds(cbase, nchunk)], idxp_v)
    pltpu.sync_copy(idxe_hbm.at[pl.ds(cbase, nchunk)], idxe_v)

    def start(c, slot):
        pltpu.async_copy(magno_hbm.at[idxp_v.at[c]], rows_v[slot], gsems[slot])
        pltpu.async_copy(pos_hbm.at[idxe_v.at[c]], pose_v[slot], gsems[slot])

    def finish(c, slot):
        pltpu.make_async_copy(magno_hbm.at[idxp_v.at[c]], rows_v[slot],
                              gsems[slot]).wait()
        pltpu.make_async_copy(pos_hbm.at[idxe_v.at[c]], pose_v[slot],
                              gsems[slot]).wait()

        def addrow(r, carry):
            for d0 in range(0, D, 16):
                rows_v[slot][r, pl.ds(d0, 16)] = (
                    rows_v[slot][r, pl.ds(d0, 16)]
                    + pose_v[slot][r, pl.ds(d0, 16)])
            return carry

        lax.fori_loop(0, _CHUNK, addrow, 0)
        copy = pltpu.async_copy(rows_v[slot], out_slice(c), ssems[slot])
        return copy

    start(0, 0)

    def group(g, carry):
        for b in range(_NBUF):
            c = g * _NBUF + b
            nslot = (b + 1) % _NBUF

            @pl.when(c + 1 < nchunk)
            def _():
                @pl.when(c + 1 >= _NBUF)
                def _():
                    pltpu.make_async_copy(rows_v[nslot],
                                          out_slice(c + 1 - _NBUF),
                                          ssems[nslot]).wait()
                start(c + 1, nslot)

            finish(c, b)
        return carry

    lax.fori_loop(0, nchunk // _NBUF, group, 0)
    for b in range(_NBUF):
        c = nchunk - _NBUF + b
        slot = c % _NBUF
        pltpu.make_async_copy(rows_v[slot], out_slice(c),
                              ssems[slot]).wait()


def _sc_gather(magno_flat, pos_flat, idxp, idxe, B, K, D):
    nchunk_w = B * K // _NW // _CHUNK
    mesh = plsc.VectorSubcoreMesh(core_axis_name="c", subcore_axis_name="s")
    return pl.kernel(
        _sc_gather_body,
        out_type=jax.ShapeDtypeStruct((B, K, D), jnp.float32),
        mesh=mesh,
        scratch_types=[
            pltpu.VMEM((nchunk_w, _CHUNK), jnp.int32),
            pltpu.VMEM((nchunk_w, _CHUNK), jnp.int32),
            [pltpu.VMEM((_CHUNK, D), jnp.float32) for _ in range(_NBUF)],
            [pltpu.VMEM((_CHUNK, D), jnp.float32) for _ in range(_NBUF)],
            [pltpu.SemaphoreType.DMA for _ in range(_NBUF)],
            [pltpu.SemaphoreType.DMA for _ in range(_NBUF)],
        ],
    )(magno_flat, pos_flat, idxp, idxe)


def kernel(magno_patches, vit_positional_embedding, scores):
    B, N, D = magno_patches.shape
    K = N // 4
    idxp, idxe = _topk_indices(scores, B, N, K)
    magno_flat = magno_patches.reshape(B * N, D)
    pos_flat = vit_positional_embedding[0]
    return _sc_gather(magno_flat, pos_flat, idxp, idxe, B, K, D)


# speedup vs baseline: 1.0004x; 1.0004x over previous
import functools

import jax
import jax.numpy as jnp
from jax import lax
from jax.experimental import pallas as pl
from jax.experimental.pallas import tpu as pltpu
from jax.experimental.pallas import tpu_sc as plsc


def _topk_idx_body(scores_ref, idxp_ref, idxe_ref):
    N = scores_ref.shape[-1]
    K = idxp_ref.shape[-1]
    b = pl.program_id(0)
    s = scores_ref[0]
    scol = jnp.reshape(s, (N, 1))
    ni = lax.broadcasted_iota(jnp.int32, (N, N), 0)
    mi = lax.broadcasted_iota(jnp.int32, (N, N), 1)
    beats = (s > scol) | ((s == scol) & (mi < ni))
    ones = jnp.ones((N, 8), jnp.float32)
    rank_f = lax.dot_general(beats.astype(jnp.float32), ones,
                             dimension_numbers=(((1,), (0,)), ((), ())))
    rank = rank_f[:, :1].astype(jnp.int32)
    jrow = lax.broadcasted_iota(jnp.int32, (1, K), 1)
    sel = rank == jrow
    nidx = lax.broadcasted_iota(jnp.int32, (N, K), 0)
    idx = jnp.sum(jnp.where(sel, nidx, 0), axis=0, keepdims=True)
    idxp_ref[0] = idx + b * N
    idxe_ref[0] = idx + 1


def _topk_indices(scores, B, N, K):
    scores3 = scores.reshape(B, 1, N)
    idxp, idxe = pl.pallas_call(
        _topk_idx_body,
        grid=(B,),
        in_specs=[pl.BlockSpec((1, 1, N), lambda b: (b, 0, 0))],
        out_specs=[
            pl.BlockSpec((1, 1, K), lambda b: (b, 0, 0)),
            pl.BlockSpec((1, 1, K), lambda b: (b, 0, 0)),
        ],
        out_shape=[
            jax.ShapeDtypeStruct((B, 1, K), jnp.int32),
            jax.ShapeDtypeStruct((B, 1, K), jnp.int32),
        ],
    )(scores3)
    return (idxp.reshape(B * K // _CHUNK, _CHUNK),
            idxe.reshape(B * K // _CHUNK, _CHUNK))


_NC, _NS = 2, 16
_NW = _NC * _NS
_CHUNK = 16
_NBUF = 4


def _sc_gather_body(magno_hbm, pos_hbm, idxp_hbm, idxe_hbm, out_hbm,
                    idxp_v, idxe_v, rows_v, pose_v, gsems, ssems):
    D = rows_v[0].shape[-1]
    K = out_hbm.shape[1]
    rows_total = out_hbm.shape[0] * K
    rows_per_w = rows_total // _NW

    def out_slice(c):
        off = base + c * _CHUNK
        return out_hbm.at[off // K, pl.ds(pl.multiple_of(off % K, _CHUNK),
                                          _CHUNK)]
    nchunk = rows_per_w // _CHUNK
    wid = lax.axis_index("s") * _NC + lax.axis_index("c")
    base = wid * rows_per_w
    cbase = wid * nchunk

    pltpu.sync_copy(idxp_hbm.at[pl.ds(cbase, nchunk)], idxp_v)
    pltpu.sync_copy(idxe_hbm.at[pl.ds(cbase, nchunk)], idxe_v)

    def start(c, slot):
        pltpu.async_copy(magno_hbm.at[idxp_v.at[c]], rows_v[slot], gsems[slot])
        pltpu.async_copy(pos_hbm.at[idxe_v.at[c]], pose_v[slot], gsems[slot])

    def finish(c, slot):
        pltpu.make_async_copy(magno_hbm.at[idxp_v.at[c]], rows_v[slot],
                              gsems[slot]).wait()
        pltpu.make_async_copy(pos_hbm.at[idxe_v.at[c]], pose_v[slot],
                              gsems[slot]).wait()

        def addrow(r, carry):
            for d0 in range(0, D, 16):
                rows_v[slot][r, pl.ds(d0, 16)] = (
                    rows_v[slot][r, pl.ds(d0, 16)]
                    + pose_v[slot][r, pl.ds(d0, 16)])
            return carry

        lax.fori_loop(0, _CHUNK, addrow, 0)
        copy = pltpu.async_copy(rows_v[slot], out_slice(c), ssems[slot])
        return copy

    _AHEAD = _NBUF - 1
    for a in range(_AHEAD):
        start(a, a)

    def group(g, carry):
        for b in range(_NBUF):
            c = g * _NBUF + b
            nslot = (b + _AHEAD) % _NBUF

            @pl.when(c + _AHEAD < nchunk)
            def _():
                @pl.when(c + _AHEAD >= _NBUF)
                def _():
                    pltpu.make_async_copy(rows_v[nslot],
                                          out_slice(c + _AHEAD - _NBUF),
                                          ssems[nslot]).wait()
                start(c + _AHEAD, nslot)

            finish(c, b)
        return carry

    lax.fori_loop(0, nchunk // _NBUF, group, 0)
    for b in range(_NBUF):
        c = nchunk - _NBUF + b
        slot = c % _NBUF
        pltpu.make_async_copy(rows_v[slot], out_slice(c),
                              ssems[slot]).wait()


def _sc_gather(magno_flat, pos_flat, idxp, idxe, B, K, D):
    nchunk_w = B * K // _NW // _CHUNK
    mesh = plsc.VectorSubcoreMesh(core_axis_name="c", subcore_axis_name="s")
    return pl.kernel(
        _sc_gather_body,
        out_type=jax.ShapeDtypeStruct((B, K, D), jnp.float32),
        mesh=mesh,
        scratch_types=[
            pltpu.VMEM((nchunk_w, _CHUNK), jnp.int32),
            pltpu.VMEM((nchunk_w, _CHUNK), jnp.int32),
            [pltpu.VMEM((_CHUNK, D), jnp.float32) for _ in range(_NBUF)],
            [pltpu.VMEM((_CHUNK, D), jnp.float32) for _ in range(_NBUF)],
            [pltpu.SemaphoreType.DMA for _ in range(_NBUF)],
            [pltpu.SemaphoreType.DMA for _ in range(_NBUF)],
        ],
    )(magno_flat, pos_flat, idxp, idxe)


def kernel(magno_patches, vit_positional_embedding, scores):
    B, N, D = magno_patches.shape
    K = N // 4
    idxp, idxe = _topk_indices(scores, B, N, K)
    magno_flat = magno_patches.reshape(B * N, D)
    pos_flat = vit_positional_embedding[0]
    return _sc_gather(magno_flat, pos_flat, idxp, idxe, B, K, D)


# speedup vs baseline: 1.0603x; 1.0598x over previous
import functools

import jax
import jax.numpy as jnp
from jax import lax
from jax.experimental import pallas as pl
from jax.experimental.pallas import tpu as pltpu
from jax.experimental.pallas import tpu_sc as plsc


_RPS = 2


def _topk_idx_body(scores_ref, idxp_ref, idxe_ref):
    N = scores_ref.shape[-1]
    K = idxp_ref.shape[-1]
    g = pl.program_id(0)
    for r in range(_RPS):
        b = g * _RPS + r
        s = scores_ref[r]
        scol = jnp.reshape(s, (N, 1))
        ni = lax.broadcasted_iota(jnp.int32, (N, N), 0)
        mi = lax.broadcasted_iota(jnp.int32, (N, N), 1)
        beats = (s > scol) | ((s == scol) & (mi < ni))
        ones = jnp.ones((N, 8), jnp.float32)
        rank_f = lax.dot_general(beats.astype(jnp.float32), ones,
                                 dimension_numbers=(((1,), (0,)), ((), ())))
        rank = rank_f[:, :1].astype(jnp.int32)
        jrow = lax.broadcasted_iota(jnp.int32, (1, K), 1)
        sel = rank == jrow
        nidx = lax.broadcasted_iota(jnp.int32, (N, K), 0)
        idx = jnp.sum(jnp.where(sel, nidx, 0), axis=0, keepdims=True)
        idxp_ref[r] = idx + b * N
        idxe_ref[r] = idx + 1


def _topk_indices(scores, B, N, K):
    scores3 = scores.reshape(B, 1, N)
    idxp, idxe = pl.pallas_call(
        _topk_idx_body,
        grid=(B // _RPS,),
        in_specs=[pl.BlockSpec((_RPS, 1, N), lambda g: (g, 0, 0))],
        out_specs=[
            pl.BlockSpec((_RPS, 1, K), lambda g: (g, 0, 0)),
            pl.BlockSpec((_RPS, 1, K), lambda g: (g, 0, 0)),
        ],
        out_shape=[
            jax.ShapeDtypeStruct((B, 1, K), jnp.int32),
            jax.ShapeDtypeStruct((B, 1, K), jnp.int32),
        ],
    )(scores3)
    return (idxp.reshape(B * K // _CHUNK, _CHUNK),
            idxe.reshape(B * K // _CHUNK, _CHUNK))


_NC, _NS = 2, 16
_NW = _NC * _NS
_CHUNK = 16
_NBUF = 4


def _sc_gather_body(magno_hbm, pos_hbm, idxp_hbm, idxe_hbm, out_hbm,
                    idxp_v, idxe_v, rows_v, pose_v, gsems, ssems):
    D = rows_v[0].shape[-1]
    K = out_hbm.shape[1]
    rows_total = out_hbm.shape[0] * K
    rows_per_w = rows_total // _NW

    def out_slice(c):
        off = base + c * _CHUNK
        return out_hbm.at[off // K, pl.ds(pl.multiple_of(off % K, _CHUNK),
                                          _CHUNK)]
    nchunk = rows_per_w // _CHUNK
    wid = lax.axis_index("s") * _NC + lax.axis_index("c")
    base = wid * rows_per_w
    cbase = wid * nchunk

    pltpu.sync_copy(idxp_hbm.at[pl.ds(cbase, nchunk)], idxp_v)
    pltpu.sync_copy(idxe_hbm.at[pl.ds(cbase, nchunk)], idxe_v)

    def start(c, slot):
        pltpu.async_copy(magno_hbm.at[idxp_v.at[c]], rows_v[slot], gsems[slot])
        pltpu.async_copy(pos_hbm.at[idxe_v.at[c]], pose_v[slot], gsems[slot])

    def finish(c, slot):
        pltpu.make_async_copy(magno_hbm.at[idxp_v.at[c]], rows_v[slot],
                              gsems[slot]).wait()
        pltpu.make_async_copy(pos_hbm.at[idxe_v.at[c]], pose_v[slot],
                              gsems[slot]).wait()

        def addrow(r, carry):
            for d0 in range(0, D, 16):
                rows_v[slot][r, pl.ds(d0, 16)] = (
                    rows_v[slot][r, pl.ds(d0, 16)]
                    + pose_v[slot][r, pl.ds(d0, 16)])
            return carry

        lax.fori_loop(0, _CHUNK, addrow, 0)
        copy = pltpu.async_copy(rows_v[slot], out_slice(c), ssems[slot])
        return copy

    _AHEAD = _NBUF - 1
    for a in range(_AHEAD):
        start(a, a)

    def group(g, carry):
        for b in range(_NBUF):
            c = g * _NBUF + b
            nslot = (b + _AHEAD) % _NBUF

            @pl.when(c + _AHEAD < nchunk)
            def _():
                @pl.when(c + _AHEAD >= _NBUF)
                def _():
                    pltpu.make_async_copy(rows_v[nslot],
                                          out_slice(c + _AHEAD - _NBUF),
                                          ssems[nslot]).wait()
                start(c + _AHEAD, nslot)

            finish(c, b)
        return carry

    lax.fori_loop(0, nchunk // _NBUF, group, 0)
    for b in range(_NBUF):
        c = nchunk - _NBUF + b
        slot = c % _NBUF
        pltpu.make_async_copy(rows_v[slot], out_slice(c),
                              ssems[slot]).wait()


def _sc_gather(magno_flat, pos_flat, idxp, idxe, B, K, D):
    nchunk_w = B * K // _NW // _CHUNK
    mesh = plsc.VectorSubcoreMesh(core_axis_name="c", subcore_axis_name="s")
    return pl.kernel(
        _sc_gather_body,
        out_type=jax.ShapeDtypeStruct((B, K, D), jnp.float32),
        mesh=mesh,
        scratch_types=[
            pltpu.VMEM((nchunk_w, _CHUNK), jnp.int32),
            pltpu.VMEM((nchunk_w, _CHUNK), jnp.int32),
            [pltpu.VMEM((_CHUNK, D), jnp.float32) for _ in range(_NBUF)],
            [pltpu.VMEM((_CHUNK, D), jnp.float32) for _ in range(_NBUF)],
            [pltpu.SemaphoreType.DMA for _ in range(_NBUF)],
            [pltpu.SemaphoreType.DMA for _ in range(_NBUF)],
        ],
    )(magno_flat, pos_flat, idxp, idxe)


def kernel(magno_patches, vit_positional_embedding, scores):
    B, N, D = magno_patches.shape
    K = N // 4
    idxp, idxe = _topk_indices(scores, B, N, K)
    magno_flat = magno_patches.reshape(B * N, D)
    pos_flat = vit_positional_embedding[0]
    return _sc_gather(magno_flat, pos_flat, idxp, idxe, B, K, D)


# speedup vs baseline: 1.1008x; 1.0382x over previous
import functools

import jax
import jax.numpy as jnp
from jax import lax
from jax.experimental import pallas as pl
from jax.experimental.pallas import tpu as pltpu
from jax.experimental.pallas import tpu_sc as plsc


_RPS = 4


def _topk_idx_body(scores_ref, idxp_ref, idxe_ref):
    N = scores_ref.shape[-1]
    K = idxp_ref.shape[-1]
    g = pl.program_id(0)
    for r in range(_RPS):
        b = g * _RPS + r
        s = scores_ref[r]
        scol = jnp.reshape(s, (N, 1))
        ni = lax.broadcasted_iota(jnp.int32, (N, N), 0)
        mi = lax.broadcasted_iota(jnp.int32, (N, N), 1)
        beats = (s > scol) | ((s == scol) & (mi < ni))
        ones = jnp.ones((N, 8), jnp.float32)
        rank_f = lax.dot_general(beats.astype(jnp.float32), ones,
                                 dimension_numbers=(((1,), (0,)), ((), ())))
        rank = rank_f[:, :1].astype(jnp.int32)
        jrow = lax.broadcasted_iota(jnp.int32, (1, K), 1)
        sel = rank == jrow
        nidx = lax.broadcasted_iota(jnp.int32, (N, K), 0)
        idx = jnp.sum(jnp.where(sel, nidx, 0), axis=0, keepdims=True)
        idxp_ref[r] = idx + b * N
        idxe_ref[r] = idx + 1


def _topk_indices(scores, B, N, K):
    scores3 = scores.reshape(B, 1, N)
    idxp, idxe = pl.pallas_call(
        _topk_idx_body,
        grid=(B // _RPS,),
        in_specs=[pl.BlockSpec((_RPS, 1, N), lambda g: (g, 0, 0))],
        out_specs=[
            pl.BlockSpec((_RPS, 1, K), lambda g: (g, 0, 0)),
            pl.BlockSpec((_RPS, 1, K), lambda g: (g, 0, 0)),
        ],
        out_shape=[
            jax.ShapeDtypeStruct((B, 1, K), jnp.int32),
            jax.ShapeDtypeStruct((B, 1, K), jnp.int32),
        ],
    )(scores3)
    return (idxp.reshape(B * K // _CHUNK, _CHUNK),
            idxe.reshape(B * K // _CHUNK, _CHUNK))


_NC, _NS = 2, 16
_NW = _NC * _NS
_CHUNK = 16
_NBUF = 4


def _sc_gather_body(magno_hbm, pos_hbm, idxp_hbm, idxe_hbm, out_hbm,
                    idxp_v, idxe_v, rows_v, pose_v, gsems, ssems):
    D = rows_v[0].shape[-1]
    K = out_hbm.shape[1]
    rows_total = out_hbm.shape[0] * K
    rows_per_w = rows_total // _NW

    def out_slice(c):
        off = base + c * _CHUNK
        return out_hbm.at[off // K, pl.ds(pl.multiple_of(off % K, _CHUNK),
                                          _CHUNK)]
    nchunk = rows_per_w // _CHUNK
    wid = lax.axis_index("s") * _NC + lax.axis_index("c")
    base = wid * rows_per_w
    cbase = wid * nchunk

    pltpu.sync_copy(idxp_hbm.at[pl.ds(cbase, nchunk)], idxp_v)
    pltpu.sync_copy(idxe_hbm.at[pl.ds(cbase, nchunk)], idxe_v)

    def start(c, slot):
        pltpu.async_copy(magno_hbm.at[idxp_v.at[c]], rows_v[slot], gsems[slot])
        pltpu.async_copy(pos_hbm.at[idxe_v.at[c]], pose_v[slot], gsems[slot])

    def finish(c, slot):
        pltpu.make_async_copy(magno_hbm.at[idxp_v.at[c]], rows_v[slot],
                              gsems[slot]).wait()
        pltpu.make_async_copy(pos_hbm.at[idxe_v.at[c]], pose_v[slot],
                              gsems[slot]).wait()

        def addrow(r, carry):
            for d0 in range(0, D, 16):
                rows_v[slot][r, pl.ds(d0, 16)] = (
                    rows_v[slot][r, pl.ds(d0, 16)]
                    + pose_v[slot][r, pl.ds(d0, 16)])
            return carry

        lax.fori_loop(0, _CHUNK, addrow, 0)
        copy = pltpu.async_copy(rows_v[slot], out_slice(c), ssems[slot])
        return copy

    _AHEAD = _NBUF - 1
    for a in range(_AHEAD):
        start(a, a)

    def group(g, carry):
        for b in range(_NBUF):
            c = g * _NBUF + b
            nslot = (b + _AHEAD) % _NBUF

            @pl.when(c + _AHEAD < nchunk)
            def _():
                @pl.when(c + _AHEAD >= _NBUF)
                def _():
                    pltpu.make_async_copy(rows_v[nslot],
                                          out_slice(c + _AHEAD - _NBUF),
                                          ssems[nslot]).wait()
                start(c + _AHEAD, nslot)

            finish(c, b)
        return carry

    lax.fori_loop(0, nchunk // _NBUF, group, 0)
    for b in range(_NBUF):
        c = nchunk - _NBUF + b
        slot = c % _NBUF
        pltpu.make_async_copy(rows_v[slot], out_slice(c),
                              ssems[slot]).wait()


def _sc_gather(magno_flat, pos_flat, idxp, idxe, B, K, D):
    nchunk_w = B * K // _NW // _CHUNK
    mesh = plsc.VectorSubcoreMesh(core_axis_name="c", subcore_axis_name="s")
    return pl.kernel(
        _sc_gather_body,
        out_type=jax.ShapeDtypeStruct((B, K, D), jnp.float32),
        mesh=mesh,
        scratch_types=[
            pltpu.VMEM((nchunk_w, _CHUNK), jnp.int32),
            pltpu.VMEM((nchunk_w, _CHUNK), jnp.int32),
            [pltpu.VMEM((_CHUNK, D), jnp.float32) for _ in range(_NBUF)],
            [pltpu.VMEM((_CHUNK, D), jnp.float32) for _ in range(_NBUF)],
            [pltpu.SemaphoreType.DMA for _ in range(_NBUF)],
            [pltpu.SemaphoreType.DMA for _ in range(_NBUF)],
        ],
    )(magno_flat, pos_flat, idxp, idxe)


def kernel(magno_patches, vit_positional_embedding, scores):
    B, N, D = magno_patches.shape
    K = N // 4
    idxp, idxe = _topk_indices(scores, B, N, K)
    magno_flat = magno_patches.reshape(B * N, D)
    pos_flat = vit_positional_embedding[0]
    return _sc_gather(magno_flat, pos_flat, idxp, idxe, B, K, D)
